# baseline (speedup 1.0000x reference)
import math

import jax
import jax.numpy as jnp
from jax.experimental import pallas as pl
from jax.experimental.pallas import tpu as pltpu

WP = 10
WF = 10
EPS = 1e-8
NEG = -1e30
NC = 8


def _att_kernel(tl_ref, nf_ref, kn_ref, an_ref, ws_ref, wc_ref, out_ref):
    b = pl.program_id(0)
    h = pl.program_id(1)
    nh = pl.num_programs(1)
    L = nf_ref.shape[1]
    D = kn_ref.shape[3]

    k4 = kn_ref[0]
    kt = jnp.transpose(k4, (1, 0, 2))
    k2 = kt.reshape(NC * L, D)
    c2 = jax.lax.dot_general(k2, wc_ref[...], (((1,), (0,)), ((), ())),
                             preferred_element_type=jnp.float32)
    k3 = kt
    c3 = c2.reshape(NC, L, D)

    nk2 = jnp.sum(k3 * k3, axis=2)
    nw2 = jnp.sum(c3 * c3, axis=2)
    va = jnp.transpose(an_ref[0, :, :, 0])
    aff = (jnp.sqrt((va - 0.5) ** 2 + (0.5 * va) ** 2) - 0.06467) / 0.607468
    a_abs = jnp.abs(aff)
    w = a_abs / jnp.maximum(a_abs * jnp.sqrt(nw2), EPS)
    u = 1.0 / jnp.maximum(jnp.sqrt(nk2), EPS)
    ut = jnp.transpose(u)

    acc = jnp.zeros((L, L), jnp.float32)
    for i in range(NC):
        sn = jax.lax.dot_general(k3[i], c3[i], (((1,), (1,)), ((), ())),
                                 preferred_element_type=jnp.float32)
        acc = acc + jnp.abs(sn) * (w[i:i + 1, :] * ut[:, i:i + 1])

    @pl.when(h == 0)
    def _init():
        out_ref[0] = jnp.zeros((L, L), jnp.float32)

    out_ref[0] = out_ref[0] + acc

    @pl.when(h == nh - 1)
    def _finalize():
        nf = nf_ref[0]
        att = jax.lax.dot_general(nf, ws_ref[...], (((1,), (1,)), ((), ())),
                                  preferred_element_type=jnp.float32)
        dots = jax.lax.dot_general(nf, att, (((1,), (1,)), ((), ())),
                                   preferred_element_type=jnp.float32)
        n_nf = jnp.maximum(jnp.sqrt(jnp.sum(nf * nf, axis=1, keepdims=True)), EPS)
        n_at = jnp.maximum(jnp.sqrt(jnp.sum(att * att, axis=1, keepdims=True)), EPS)
        cos = dots / (n_nf * jnp.transpose(n_at))
        cos = jnp.clip(cos, -1.0 + 1e-6, 1.0 - 1e-6)
        ax = jnp.abs(cos)
        p = jnp.float32(-0.0012624911)
        for c in (0.0066700901, -0.0170881256, 0.0308918810, -0.0501743046,
                  0.0889789874, -0.2145988016, 1.5707963050):
            p = p * ax + jnp.float32(c)
        r = jnp.sqrt(1.0 - ax) * p
        acos = jnp.where(cos >= 0.0, r, jnp.float32(math.pi) - r)
        score = 1.0 - acos / math.pi

        cur = tl_ref[b]
        jj = jax.lax.broadcasted_iota(jnp.int32, (L, L), 0)
        kk = jax.lax.broadcasted_iota(jnp.int32, (L, L), 1)
        mask = (kk >= jj - WP) & (kk <= jj + WF) & (jj < cur) & (kk <= cur - 1)

        mx = jnp.max(jnp.where(mask, score, -jnp.inf), axis=1, keepdims=True)
        mx = jnp.where(jnp.isinf(mx), 0.0, mx)
        sh = jnp.where(mask, score - mx, NEG)
        ex = jnp.exp(sh)
        den = jnp.sum(ex, axis=1, keepdims=True)
        alphas_sem = jnp.where(den > 0.0, ex / jnp.where(den > 0.0, den, 1.0), 0.0)

        out_ref[0] = 0.5 * alphas_sem + 5.0 * jnp.where(mask, out_ref[0], 0.0)


def kernel(node_features, text_len_tensor, knowledge, anew, edge_ind, weight_sem, weight_con):
    del edge_ind
    B, L, G = node_features.shape
    _, _, N, D = knowledge.shape
    assert N % NC == 0
    nh = N // NC

    tl = text_len_tensor.astype(jnp.int32)

    grid = (B, nh)
    grid_spec = pltpu.PrefetchScalarGridSpec(
        num_scalar_prefetch=1,
        grid=grid,
        in_specs=[
            pl.BlockSpec((1, L, G), lambda b, h, tl_ref: (b, 0, 0)),
            pl.BlockSpec((1, L, NC, D), lambda b, h, tl_ref: (b, 0, h, 0)),
            pl.BlockSpec((1, L, NC, 1), lambda b, h, tl_ref: (b, 0, h, 0)),
            pl.BlockSpec((G, G), lambda b, h, tl_ref: (0, 0)),
            pl.BlockSpec((D, D), lambda b, h, tl_ref: (0, 0)),
        ],
        out_specs=pl.BlockSpec((1, L, L), lambda b, h, tl_ref: (b, 0, 0)),
    )
    return pl.pallas_call(
        _att_kernel,
        grid_spec=grid_spec,
        out_shape=jax.ShapeDtypeStruct((B, L, L), jnp.float32),
        compiler_params=pltpu.CompilerParams(
            dimension_semantics=("arbitrary", "arbitrary"),
        ),
    )(tl, node_features, knowledge, anew, weight_sem, weight_con)


# speedup vs baseline: 1.1193x; 1.1193x over previous
import math

import jax
import jax.numpy as jnp
from jax.experimental import pallas as pl
from jax.experimental.pallas import tpu as pltpu

WP = 10
WF = 10
EPS = 1e-8
NEG = -1e30
NC = 40


def _att_kernel(tl_ref, nf_ref, kn_ref, an_ref, ws_ref, wc_ref, out_ref):
    b = pl.program_id(0)
    h = pl.program_id(1)
    nh = pl.num_programs(1)
    L = nf_ref.shape[1]
    D = kn_ref.shape[3]

    k4 = kn_ref[0]
    kt = jnp.transpose(k4, (1, 0, 2))
    k2 = kt.reshape(NC * L, D)
    c2 = jax.lax.dot_general(k2, wc_ref[...], (((1,), (0,)), ((), ())),
                             preferred_element_type=jnp.float32)
    k3 = kt
    c3 = c2.reshape(NC, L, D)

    nk2 = jnp.sum(k3 * k3, axis=2)
    nw2 = jnp.sum(c3 * c3, axis=2)
    va = an_ref[0, 0]
    aff = (jnp.sqrt((va - 0.5) ** 2 + (0.5 * va) ** 2) - 0.06467) / 0.607468
    a_abs = jnp.abs(aff)
    w = a_abs / jnp.maximum(a_abs * jnp.sqrt(nw2), EPS)
    u = 1.0 / jnp.maximum(jnp.sqrt(nk2), EPS)
    ut = jnp.transpose(u)

    acc = jnp.zeros((L, L), jnp.float32)
    for i in range(NC):
        sn = jax.lax.dot_general(k3[i], c3[i], (((1,), (1,)), ((), ())),
                                 preferred_element_type=jnp.float32)
        acc = acc + jnp.abs(sn) * (w[i:i + 1, :] * ut[:, i:i + 1])

    @pl.when(h == 0)
    def _init():
        out_ref[0] = jnp.zeros((L, L), jnp.float32)

    out_ref[0] = out_ref[0] + acc

    @pl.when(h == nh - 1)
    def _finalize():
        nf = nf_ref[0]
        att = jax.lax.dot_general(nf, ws_ref[...], (((1,), (1,)), ((), ())),
                                  preferred_element_type=jnp.float32)
        dots = jax.lax.dot_general(nf, att, (((1,), (1,)), ((), ())),
                                   preferred_element_type=jnp.float32)
        n_nf = jnp.maximum(jnp.sqrt(jnp.sum(nf * nf, axis=1, keepdims=True)), EPS)
        n_at = jnp.maximum(jnp.sqrt(jnp.sum(att * att, axis=1, keepdims=True)), EPS)
        cos = dots / (n_nf * jnp.transpose(n_at))
        cos = jnp.clip(cos, -1.0 + 1e-6, 1.0 - 1e-6)
        ax = jnp.abs(cos)
        p = jnp.float32(-0.0012624911)
        for c in (0.0066700901, -0.0170881256, 0.0308918810, -0.0501743046,
                  0.0889789874, -0.2145988016, 1.5707963050):
            p = p * ax + jnp.float32(c)
        r = jnp.sqrt(1.0 - ax) * p
        acos = jnp.where(cos >= 0.0, r, jnp.float32(math.pi) - r)
        score = 1.0 - acos / math.pi

        cur = tl_ref[b]
        jj = jax.lax.broadcasted_iota(jnp.int32, (L, L), 0)
        kk = jax.lax.broadcasted_iota(jnp.int32, (L, L), 1)
        mask = (kk >= jj - WP) & (kk <= jj + WF) & (jj < cur) & (kk <= cur - 1)

        mx = jnp.max(jnp.where(mask, score, -jnp.inf), axis=1, keepdims=True)
        mx = jnp.where(jnp.isinf(mx), 0.0, mx)
        sh = jnp.where(mask, score - mx, NEG)
        ex = jnp.exp(sh)
        den = jnp.sum(ex, axis=1, keepdims=True)
        alphas_sem = jnp.where(den > 0.0, ex / jnp.where(den > 0.0, den, 1.0), 0.0)

        out_ref[0] = 0.5 * alphas_sem + 5.0 * jnp.where(mask, out_ref[0], 0.0)


def kernel(node_features, text_len_tensor, knowledge, anew, edge_ind, weight_sem, weight_con):
    del edge_ind
    B, L, G = node_features.shape
    _, _, N, D = knowledge.shape
    assert N % NC == 0
    nh = N // NC

    an_t = jnp.transpose(anew[..., 0], (0, 2, 1)).reshape(B, nh, NC, L)
    tl = text_len_tensor.astype(jnp.int32)

    grid = (B, nh)
    grid_spec = pltpu.PrefetchScalarGridSpec(
        num_scalar_prefetch=1,
        grid=grid,
        in_specs=[
            pl.BlockSpec((1, L, G), lambda b, h, tl_ref: (b, 0, 0)),
            pl.BlockSpec((1, L, NC, D), lambda b, h, tl_ref: (b, 0, h, 0)),
            pl.BlockSpec((1, 1, NC, L), lambda b, h, tl_ref: (b, h, 0, 0)),
            pl.BlockSpec((G, G), lambda b, h, tl_ref: (0, 0)),
            pl.BlockSpec((D, D), lambda b, h, tl_ref: (0, 0)),
        ],
        out_specs=pl.BlockSpec((1, L, L), lambda b, h, tl_ref: (b, 0, 0)),
    )
    return pl.pallas_call(
        _att_kernel,
        grid_spec=grid_spec,
        out_shape=jax.ShapeDtypeStruct((B, L, L), jnp.float32),
        compiler_params=pltpu.CompilerParams(
            dimension_semantics=("arbitrary", "arbitrary"),
        ),
    )(tl, node_features, knowledge, an_t, weight_sem, weight_con)
